# Initial kernel scaffold; baseline (speedup 1.0000x reference)
import jax
import jax.numpy as jnp
from jax.experimental import pallas as pl


def kernel(f, h, x_T, x_G, edge_index, graph_ids, wz, uz, bz, wr, ur, br, wg, ug, bg, wd1, wd2, bd1, a_t, a_g, wd3, wd4, bd2, ud, bd3):
    raise NotImplementedError("write your pallas kernel here")


# speedup vs baseline: 7.8198x; 7.8198x over previous
import functools

import jax
import jax.numpy as jnp
from jax import lax
from jax.experimental import pallas as pl
from jax.experimental.pallas import tpu as pltpu
from jax.experimental.pallas import tpu_sc as plsc

M = 256
NEG = -3.0e38


def _sc_scatter(h, rh, src3, dst3, *, n, d, ns, nch, k):
    mesh = plsc.VectorSubcoreMesh(
        core_axis_name="c", subcore_axis_name="s", num_cores=2,
        num_subcores=ns)
    zc = 8
    np_ = ((n + ns * 128 - 1) // (ns * 128)) * ns * 128
    zr = np_ // ns
    nz = zr // zc
    sc_ = 16
    nsc = nch // sc_

    @functools.partial(
        pl.kernel,
        out_type=[jax.ShapeDtypeStruct((np_, d), jnp.float32),
                  jax.ShapeDtypeStruct((np_, d), jnp.float32)],
        mesh=mesh,
        scratch_types=[
            pltpu.VMEM((sc_, k), jnp.int32),
            pltpu.VMEM((sc_, k), jnp.int32),
            pltpu.VMEM((k, d), jnp.float32),
            pltpu.VMEM((zc, d), jnp.float32),
            pltpu.VMEM_SHARED((np_, d), jnp.float32),
            pltpu.SemaphoreType.DMA,
        ],
    )
    def scatter_kernel(h_hbm, rh_hbm, src_hbm, dst_hbm, s_hbm, srh_hbm,
                       src_v, dst_v, rows_v, zbuf_v, acc_sh, sem):
        c = lax.axis_index("c")
        s = lax.axis_index("s")
        def zrow(t, carry):
            i = t // (d // 16)
            j = t % (d // 16)
            zbuf_v[i, pl.ds(j * 16, 16)] = jnp.zeros((16,), jnp.float32)
            return carry
        lax.fori_loop(0, zc * (d // 16), zrow, 0)
        def zacc(t, carry):
            pltpu.sync_copy(zbuf_v, acc_sh.at[pl.ds(s * zr + t * zc, zc)])
            return carry
        lax.fori_loop(0, nz, zacc, 0)
        plsc.subcore_barrier()

        def edge_loop(vref):
            def super_step(t, carry):
                pltpu.sync_copy(src_hbm.at[s, t], src_v)
                pltpu.sync_copy(dst_hbm.at[s, t], dst_v)
                def step(j, carry2):
                    pltpu.async_copy(vref.at[src_v.at[j]], rows_v, sem).wait()
                    pltpu.sync_copy(rows_v, acc_sh.at[dst_v.at[j]], add=True)
                    return carry2
                lax.fori_loop(0, sc_, step, 0)
                return carry
            lax.fori_loop(0, nsc, super_step, 0)

        @pl.when(c == 0)
        def _():
            edge_loop(h_hbm)

        @pl.when(c == 1)
        def _():
            edge_loop(rh_hbm)

        plsc.subcore_barrier()

        @pl.when(c == 0)
        def _():
            pltpu.sync_copy(acc_sh.at[pl.ds(s * zr, zr)],
                            s_hbm.at[pl.ds(s * zr, zr)])

        @pl.when(c == 1)
        def _():
            pltpu.sync_copy(acc_sh.at[pl.ds(s * zr, zr)],
                            srh_hbm.at[pl.ds(s * zr, zr)])

    return scatter_kernel(h, rh, src3, dst3)


def _dot(a, b):
    return jax.lax.dot_general(a, b, (((1,), (0,)), ((), ())),
                               preferred_element_type=jnp.float32)


def _rh_kernel(f, h, wr, ur, br, *, n, d, b):
    def body(f_ref, h_ref, wr_ref, ur_ref, br_ref, o_ref):
        hb = h_ref[...]
        r = jax.nn.sigmoid(_dot(f_ref[...], wr_ref[...]) +
                           _dot(hb, ur_ref[...]) + br_ref[...])
        o_ref[...] = r * hb
    nb = n // b
    blk = lambda: pl.BlockSpec((b, d), lambda i: (i, 0))
    full = lambda r, c: pl.BlockSpec((r, c), lambda i: (0, 0))
    return pl.pallas_call(
        body,
        grid=(nb,),
        in_specs=[blk(), blk(), full(d, d), full(d, d), full(1, d)],
        out_specs=blk(),
        out_shape=jax.ShapeDtypeStruct((n, d), jnp.float32),
    )(f, h, wr, ur, br)


def _gates_kernel(f, s_arr, srh_arr, gid_row, wz, uz, bz, wg, ug, bg,
                  wd1, wd2, bd1, *, n, d, b):
    nb = n // b

    def body(f_ref, s_ref, srh_ref, g_ref, wz_ref, uz_ref, bz_ref,
             wg_ref, ug_ref, bg_ref, wd1_ref, wd2_ref, bd1_ref,
             hgsum_ref, cnt_ref):
        i = pl.program_id(0)
        fb = f_ref[...]
        sb = s_ref[...]
        srhb = srh_ref[...]
        z = jax.nn.sigmoid(_dot(fb, wz_ref[...]) + _dot(sb, uz_ref[...]) +
                           bz_ref[...])
        ht = jnp.tanh(_dot(fb, wg_ref[...]) + _dot(srhb, ug_ref[...]) +
                      bg_ref[...])
        hn = (1.0 - z) * sb + z * ht
        hd = jax.nn.relu(_dot(fb, wd1_ref[...]) + _dot(hn, wd2_ref[...]) +
                         bd1_ref[...])
        iota_m = jax.lax.broadcasted_iota(jnp.int32, (M, 1), 0)
        ohT = (iota_m == g_ref[0]).astype(jnp.float32)

        @pl.when(i == 0)
        def _():
            hgsum_ref[...] = jnp.zeros_like(hgsum_ref)
            cnt_ref[...] = jnp.zeros_like(cnt_ref)

        hgsum_ref[...] += _dot(ohT, hd)
        cnt_ref[...] += _dot(ohT, jnp.ones((b, 1), jnp.float32))

    blk = lambda im: pl.BlockSpec((b, d), im)
    full = lambda r, c: pl.BlockSpec((r, c), lambda i: (0, 0))
    return pl.pallas_call(
        body,
        grid=(nb,),
        in_specs=[
            blk(lambda i: (i, 0)),
            blk(lambda i: (i, 0)),
            blk(lambda i: (i, 0)),
            pl.BlockSpec((1, 1, b), lambda i: (i, 0, 0)),
            full(d, d), full(d, d), full(1, d),
            full(d, d), full(d, d), full(1, d),
            full(d, d), full(d, d), full(1, d),
        ],
        out_specs=[full(M, d), full(M, 1)],
        out_shape=[jax.ShapeDtypeStruct((M, d), jnp.float32),
                   jax.ShapeDtypeStruct((M, 1), jnp.float32)],
    )(f, s_arr, srh_arr, gid_row, wz, uz, bz, wg, ug, bg, wd1, wd2, bd1)


def _scores_kernel(x_T, x_G, gid_col, gid_row, hgsum, cnt, a_t, a_g,
                   *, n, d, b):
    nb = n // b

    def body(xt_ref, xg_ref, gc_ref, gr_ref, hgsum_ref, cnt_ref,
             at_ref, ag_ref, sct_ref, scg_ref, mxt_ref, mxg_ref):
        i = pl.program_id(0)
        hg = hgsum_ref[...] / jnp.maximum(cnt_ref[...], 1.0)
        q_t = _dot(hg, at_ref[...])
        q_g = _dot(hg, ag_ref[...])
        iota_m = jax.lax.broadcasted_iota(jnp.int32, (1, M), 1)
        oh = (gc_ref[...] == iota_m).astype(jnp.float32)

        @pl.when(i == 0)
        def _():
            mxt_ref[...] = jnp.full_like(mxt_ref, NEG)
            mxg_ref[...] = jnp.full_like(mxg_ref, NEG)

        def one(x_ref, q, sc_ref, mx_ref):
            sc = jnp.sum(x_ref[...] * _dot(oh, q), axis=1, keepdims=True)
            sc_ref[...] = sc
            masked = jnp.where(oh > 0.5, sc, NEG)
            mx_ref[...] = jnp.maximum(mx_ref[...],
                                      jnp.max(masked, axis=0, keepdims=True))
        one(xt_ref, q_t, sct_ref, mxt_ref)
        one(xg_ref, q_g, scg_ref, mxg_ref)

    blk = lambda: pl.BlockSpec((b, d), lambda i: (i, 0))
    full = lambda r, c: pl.BlockSpec((r, c), lambda i: (0, 0))
    return pl.pallas_call(
        body,
        grid=(nb,),
        in_specs=[
            blk(), blk(),
            pl.BlockSpec((b, 1), lambda i: (i, 0)),
            pl.BlockSpec((1, 1, b), lambda i: (i, 0, 0)),
            full(M, d), full(M, 1), full(d, d), full(d, d),
        ],
        out_specs=[pl.BlockSpec((b, 1), lambda i: (i, 0)),
                   pl.BlockSpec((b, 1), lambda i: (i, 0)),
                   full(1, M), full(1, M)],
        out_shape=[jax.ShapeDtypeStruct((n, 1), jnp.float32),
                   jax.ShapeDtypeStruct((n, 1), jnp.float32),
                   jax.ShapeDtypeStruct((1, M), jnp.float32),
                   jax.ShapeDtypeStruct((1, M), jnp.float32)],
    )(x_T, x_G, gid_col, gid_row, hgsum, cnt, a_t, a_g)


def _pool_kernel(x_T, x_G, gid_col, gid_row, sc_t, sc_g, mx_t, mx_g,
                 *, n, d, b):
    nb = n // b

    def body(xt_ref, xg_ref, gc_ref, gr_ref, sct_ref, scg_ref,
             mxt_ref, mxg_ref, dent_ref, deng_ref, ct_ref, cg_ref):
        i = pl.program_id(0)
        iota_col = jax.lax.broadcasted_iota(jnp.int32, (M, 1), 0)
        iota_row = jax.lax.broadcasted_iota(jnp.int32, (1, M), 1)
        oh = (gc_ref[...] == iota_row).astype(jnp.float32)
        ohT = (iota_col == gr_ref[0]).astype(jnp.float32)

        @pl.when(i == 0)
        def _():
            dent_ref[...] = jnp.zeros_like(dent_ref)
            deng_ref[...] = jnp.zeros_like(deng_ref)
            ct_ref[...] = jnp.zeros_like(ct_ref)
            cg_ref[...] = jnp.zeros_like(cg_ref)

        def one(x_ref, sc_ref, mx_ref, den_ref, c_ref):
            mxg = jnp.sum(oh * mx_ref[...], axis=1, keepdims=True)
            e = jnp.exp(sc_ref[...] - mxg)
            den_ref[...] += _dot(ohT, e)
            c_ref[...] += _dot(ohT, x_ref[...] * e)
        one(xt_ref, sct_ref, mxt_ref, dent_ref, ct_ref)
        one(xg_ref, scg_ref, mxg_ref, deng_ref, cg_ref)

    blk = lambda: pl.BlockSpec((b, d), lambda i: (i, 0))
    col = lambda: pl.BlockSpec((b, 1), lambda i: (i, 0))
    full = lambda r, c: pl.BlockSpec((r, c), lambda i: (0, 0))
    return pl.pallas_call(
        body,
        grid=(nb,),
        in_specs=[
            blk(), blk(),
            col(),
            pl.BlockSpec((1, 1, b), lambda i: (i, 0, 0)),
            col(), col(), full(1, M), full(1, M),
        ],
        out_specs=[full(M, 1), full(M, 1), full(M, d), full(M, d)],
        out_shape=[jax.ShapeDtypeStruct((M, 1), jnp.float32),
                   jax.ShapeDtypeStruct((M, 1), jnp.float32),
                   jax.ShapeDtypeStruct((M, d), jnp.float32),
                   jax.ShapeDtypeStruct((M, d), jnp.float32)],
    )(x_T, x_G, gid_col, gid_row, sc_t, sc_g, mx_t, mx_g)


def _final_kernel(hgsum, cnt, den_t, den_g, ct_s, cg_s,
                  wd3, wd4, bd2, ud, bd3, *, d):
    def body(hgsum_ref, cnt_ref, dent_ref, deng_ref, cts_ref, cgs_ref,
             wd3_ref, wd4_ref, bd2_ref, ud_ref, bd3_ref, o_ref):
        hg = hgsum_ref[...] / jnp.maximum(cnt_ref[...], 1.0)
        ct = cts_ref[...] / jnp.maximum(dent_ref[...], 1e-9)
        cg = cgs_ref[...] / jnp.maximum(deng_ref[...], 1e-9)
        pre = jax.nn.relu(_dot(hg, wd3_ref[...]) +
                          _dot(ct, wd4_ref[0:d, :]) +
                          _dot(cg, wd4_ref[d:2 * d, :]) + bd2_ref[...])
        o_ref[...] = _dot(pre, ud_ref[...]) + bd3_ref[...]
    return pl.pallas_call(
        body,
        out_shape=jax.ShapeDtypeStruct((M, 1), jnp.float32),
    )(hgsum, cnt, den_t, den_g, ct_s, cg_s, wd3, wd4, bd2, ud, bd3)


def kernel(f, h, x_T, x_G, edge_index, graph_ids, wz, uz, bz, wr, ur, br,
           wg, ug, bg, wd1, wd2, bd1, a_t, a_g, wd3, wd4, bd2, ud, bd3):
    n, d = f.shape
    e = edge_index.shape[1]
    ns = 16
    k = 125
    nch = e // (ns * k)
    b = 2000
    src3 = edge_index[0].astype(jnp.int32).reshape(ns, nch // 16, 16, k)
    dst3 = edge_index[1].astype(jnp.int32).reshape(ns, nch // 16, 16, k)
    gid_col = graph_ids.astype(jnp.int32).reshape(n, 1)
    gid_row = graph_ids.astype(jnp.int32).reshape(n // b, 1, b)
    bd3_2d = bd3.reshape(1, 1)

    rh = _rh_kernel(f, h, wr, ur, br, n=n, d=d, b=b)
    s_arr, srh_arr = _sc_scatter(h, rh, src3, dst3, n=n, d=d, ns=ns,
                                 nch=nch, k=k)
    hgsum, cnt = _gates_kernel(f, s_arr, srh_arr, gid_row, wz, uz, bz,
                               wg, ug, bg, wd1, wd2, bd1, n=n, d=d, b=b)
    sc_t, sc_g, mx_t, mx_g = _scores_kernel(
        x_T, x_G, gid_col, gid_row, hgsum, cnt, a_t, a_g, n=n, d=d, b=b)
    den_t, den_g, ct_s, cg_s = _pool_kernel(
        x_T, x_G, gid_col, gid_row, sc_t, sc_g, mx_t, mx_g, n=n, d=d, b=b)
    return _final_kernel(hgsum, cnt, den_t, den_g, ct_s, cg_s,
                         wd3, wd4, bd2, ud, bd3_2d, d=d)


# speedup vs baseline: 11.4538x; 1.4647x over previous
import functools

import jax
import jax.numpy as jnp
from jax import lax
from jax.experimental import pallas as pl
from jax.experimental.pallas import tpu as pltpu
from jax.experimental.pallas import tpu_sc as plsc

M = 256
NEG = -3.0e38


def _sc_scatter(h, rh, src3, dst3, *, n, d, ns, nch, k):
    mesh = plsc.VectorSubcoreMesh(
        core_axis_name="c", subcore_axis_name="s", num_cores=2,
        num_subcores=ns)
    zc = 8
    np_ = ((n + ns * 128 - 1) // (ns * 128)) * ns * 128
    zr = np_ // ns
    nz = zr // zc
    sc_ = 16
    nsc = nch // sc_

    @functools.partial(
        pl.kernel,
        out_type=[jax.ShapeDtypeStruct((np_, d), jnp.float32),
                  jax.ShapeDtypeStruct((np_, d), jnp.float32)],
        mesh=mesh,
        scratch_types=[
            pltpu.VMEM((2, sc_, k), jnp.int32),
            pltpu.VMEM((2, sc_, k), jnp.int32),
            pltpu.VMEM((k, d), jnp.float32),
            pltpu.VMEM((k, d), jnp.float32),
            pltpu.VMEM((zc, d), jnp.float32),
            pltpu.VMEM_SHARED((np_, d), jnp.float32),
            pltpu.SemaphoreType.DMA,
            pltpu.SemaphoreType.DMA,
            pltpu.SemaphoreType.DMA,
        ],
    )
    def scatter_kernel(h_hbm, rh_hbm, src_hbm, dst_hbm, s_hbm, srh_hbm,
                       src_v, dst_v, rows0_v, rows1_v, zbuf_v, acc_sh,
                       sem0, sem1, isem):
        c = lax.axis_index("c")
        s = lax.axis_index("s")
        def zrow(t, carry):
            i = t // (d // 16)
            j = t % (d // 16)
            zbuf_v[i, pl.ds(j * 16, 16)] = jnp.zeros((16,), jnp.float32)
            return carry
        lax.fori_loop(0, zc * (d // 16), zrow, 0)
        def zacc(t, carry):
            pltpu.sync_copy(zbuf_v, acc_sh.at[pl.ds(s * zr + t * zc, zc)])
            return carry
        lax.fori_loop(0, nz, zacc, 0)
        plsc.subcore_barrier()

        bufs = (rows0_v, rows1_v)
        sems = (sem0, sem1)

        def edge_loop(vref):
            pltpu.sync_copy(src_hbm.at[s, 0], src_v.at[0])
            pltpu.sync_copy(dst_hbm.at[s, 0], dst_v.at[0])

            def super_step(t, carry):
                par = lax.rem(t, 2)

                @pl.when(t > 0)
                def _():
                    pltpu.make_async_copy(src_hbm.at[s, t], src_v.at[par],
                                          isem).wait()
                    pltpu.make_async_copy(dst_hbm.at[s, t], dst_v.at[par],
                                          isem).wait()

                @pl.when(t + 1 < nsc)
                def _():
                    pltpu.async_copy(src_hbm.at[s, t + 1], src_v.at[1 - par],
                                     isem)
                    pltpu.async_copy(dst_hbm.at[s, t + 1], dst_v.at[1 - par],
                                     isem)

                cur = pltpu.async_copy(vref.at[src_v.at[par, 0]], bufs[0],
                                       sems[0])
                for i in range(sc_):
                    nxt = None
                    if i + 1 < sc_:
                        j = (i + 1) % 2
                        nxt = pltpu.async_copy(
                            vref.at[src_v.at[par, i + 1]], bufs[j], sems[j])
                    cur.wait()
                    pltpu.sync_copy(bufs[i % 2], acc_sh.at[dst_v.at[par, i]],
                                    add=True)
                    cur = nxt
                return carry
            lax.fori_loop(0, nsc, super_step, 0)

        @pl.when(c == 0)
        def _():
            edge_loop(h_hbm)

        @pl.when(c == 1)
        def _():
            edge_loop(rh_hbm)

        plsc.subcore_barrier()

        @pl.when(c == 0)
        def _():
            pltpu.sync_copy(acc_sh.at[pl.ds(s * zr, zr)],
                            s_hbm.at[pl.ds(s * zr, zr)])

        @pl.when(c == 1)
        def _():
            pltpu.sync_copy(acc_sh.at[pl.ds(s * zr, zr)],
                            srh_hbm.at[pl.ds(s * zr, zr)])

    return scatter_kernel(h, rh, src3, dst3)


def _dot(a, b):
    return jax.lax.dot_general(a, b, (((1,), (0,)), ((), ())),
                               preferred_element_type=jnp.float32)


def _rh_kernel(f, h, wr, ur, br, *, n, d, b):
    def body(f_ref, h_ref, wr_ref, ur_ref, br_ref, o_ref):
        hb = h_ref[...]
        r = jax.nn.sigmoid(_dot(f_ref[...], wr_ref[...]) +
                           _dot(hb, ur_ref[...]) + br_ref[...])
        o_ref[...] = r * hb
    nb = n // b
    blk = lambda: pl.BlockSpec((b, d), lambda i: (i, 0))
    full = lambda r, c: pl.BlockSpec((r, c), lambda i: (0, 0))
    return pl.pallas_call(
        body,
        grid=(nb,),
        in_specs=[blk(), blk(), full(d, d), full(d, d), full(1, d)],
        out_specs=blk(),
        out_shape=jax.ShapeDtypeStruct((n, d), jnp.float32),
    )(f, h, wr, ur, br)


def _gates_kernel(f, s_arr, srh_arr, gid_row, wz, uz, bz, wg, ug, bg,
                  wd1, wd2, bd1, *, n, d, b):
    nb = n // b

    def body(f_ref, s_ref, srh_ref, g_ref, wz_ref, uz_ref, bz_ref,
             wg_ref, ug_ref, bg_ref, wd1_ref, wd2_ref, bd1_ref,
             hgsum_ref, cnt_ref):
        i = pl.program_id(0)
        fb = f_ref[...]
        sb = s_ref[...]
        srhb = srh_ref[...]
        z = jax.nn.sigmoid(_dot(fb, wz_ref[...]) + _dot(sb, uz_ref[...]) +
                           bz_ref[...])
        ht = jnp.tanh(_dot(fb, wg_ref[...]) + _dot(srhb, ug_ref[...]) +
                      bg_ref[...])
        hn = (1.0 - z) * sb + z * ht
        hd = jax.nn.relu(_dot(fb, wd1_ref[...]) + _dot(hn, wd2_ref[...]) +
                         bd1_ref[...])
        iota_m = jax.lax.broadcasted_iota(jnp.int32, (M, 1), 0)
        ohT = (iota_m == g_ref[0]).astype(jnp.float32)

        @pl.when(i == 0)
        def _():
            hgsum_ref[...] = jnp.zeros_like(hgsum_ref)
            cnt_ref[...] = jnp.zeros_like(cnt_ref)

        hgsum_ref[...] += _dot(ohT, hd)
        cnt_ref[...] += _dot(ohT, jnp.ones((b, 1), jnp.float32))

    blk = lambda im: pl.BlockSpec((b, d), im)
    full = lambda r, c: pl.BlockSpec((r, c), lambda i: (0, 0))
    return pl.pallas_call(
        body,
        grid=(nb,),
        in_specs=[
            blk(lambda i: (i, 0)),
            blk(lambda i: (i, 0)),
            blk(lambda i: (i, 0)),
            pl.BlockSpec((1, 1, b), lambda i: (i, 0, 0)),
            full(d, d), full(d, d), full(1, d),
            full(d, d), full(d, d), full(1, d),
            full(d, d), full(d, d), full(1, d),
        ],
        out_specs=[full(M, d), full(M, 1)],
        out_shape=[jax.ShapeDtypeStruct((M, d), jnp.float32),
                   jax.ShapeDtypeStruct((M, 1), jnp.float32)],
    )(f, s_arr, srh_arr, gid_row, wz, uz, bz, wg, ug, bg, wd1, wd2, bd1)


def _scores_kernel(x_T, x_G, gid_col, gid_row, hgsum, cnt, a_t, a_g,
                   *, n, d, b):
    nb = n // b

    def body(xt_ref, xg_ref, gc_ref, gr_ref, hgsum_ref, cnt_ref,
             at_ref, ag_ref, sct_ref, scg_ref, mxt_ref, mxg_ref):
        i = pl.program_id(0)
        hg = hgsum_ref[...] / jnp.maximum(cnt_ref[...], 1.0)
        q_t = _dot(hg, at_ref[...])
        q_g = _dot(hg, ag_ref[...])
        iota_m = jax.lax.broadcasted_iota(jnp.int32, (1, M), 1)
        oh = (gc_ref[...] == iota_m).astype(jnp.float32)

        @pl.when(i == 0)
        def _():
            mxt_ref[...] = jnp.full_like(mxt_ref, NEG)
            mxg_ref[...] = jnp.full_like(mxg_ref, NEG)

        def one(x_ref, q, sc_ref, mx_ref):
            sc = jnp.sum(x_ref[...] * _dot(oh, q), axis=1, keepdims=True)
            sc_ref[...] = sc
            masked = jnp.where(oh > 0.5, sc, NEG)
            mx_ref[...] = jnp.maximum(mx_ref[...],
                                      jnp.max(masked, axis=0, keepdims=True))
        one(xt_ref, q_t, sct_ref, mxt_ref)
        one(xg_ref, q_g, scg_ref, mxg_ref)

    blk = lambda: pl.BlockSpec((b, d), lambda i: (i, 0))
    full = lambda r, c: pl.BlockSpec((r, c), lambda i: (0, 0))
    return pl.pallas_call(
        body,
        grid=(nb,),
        in_specs=[
            blk(), blk(),
            pl.BlockSpec((b, 1), lambda i: (i, 0)),
            pl.BlockSpec((1, 1, b), lambda i: (i, 0, 0)),
            full(M, d), full(M, 1), full(d, d), full(d, d),
        ],
        out_specs=[pl.BlockSpec((b, 1), lambda i: (i, 0)),
                   pl.BlockSpec((b, 1), lambda i: (i, 0)),
                   full(1, M), full(1, M)],
        out_shape=[jax.ShapeDtypeStruct((n, 1), jnp.float32),
                   jax.ShapeDtypeStruct((n, 1), jnp.float32),
                   jax.ShapeDtypeStruct((1, M), jnp.float32),
                   jax.ShapeDtypeStruct((1, M), jnp.float32)],
    )(x_T, x_G, gid_col, gid_row, hgsum, cnt, a_t, a_g)


def _pool_kernel(x_T, x_G, gid_col, gid_row, sc_t, sc_g, mx_t, mx_g,
                 hgsum, cnt, wd3, wd4, bd2, ud, bd3, *, n, d, b):
    nb = n // b

    def body(xt_ref, xg_ref, gc_ref, gr_ref, sct_ref, scg_ref,
             mxt_ref, mxg_ref, hgsum_ref, cnt_ref, wd3_ref, wd4_ref,
             bd2_ref, ud_ref, bd3_ref, o_ref,
             dent_ref, deng_ref, ct_ref, cg_ref):
        i = pl.program_id(0)
        iota_col = jax.lax.broadcasted_iota(jnp.int32, (M, 1), 0)
        iota_row = jax.lax.broadcasted_iota(jnp.int32, (1, M), 1)
        oh = (gc_ref[...] == iota_row).astype(jnp.float32)
        ohT = (iota_col == gr_ref[0]).astype(jnp.float32)

        @pl.when(i == 0)
        def _():
            dent_ref[...] = jnp.zeros_like(dent_ref)
            deng_ref[...] = jnp.zeros_like(deng_ref)
            ct_ref[...] = jnp.zeros_like(ct_ref)
            cg_ref[...] = jnp.zeros_like(cg_ref)

        def one(x_ref, sc_ref, mx_ref, den_ref, c_ref):
            mxg = jnp.sum(oh * mx_ref[...], axis=1, keepdims=True)
            e = jnp.exp(sc_ref[...] - mxg)
            den_ref[...] += _dot(ohT, e)
            c_ref[...] += _dot(ohT, x_ref[...] * e)
        one(xt_ref, sct_ref, mxt_ref, dent_ref, ct_ref)
        one(xg_ref, scg_ref, mxg_ref, deng_ref, cg_ref)

        @pl.when(i == nb - 1)
        def _():
            hg = hgsum_ref[...] / jnp.maximum(cnt_ref[...], 1.0)
            ct = ct_ref[...] / jnp.maximum(dent_ref[...], 1e-9)
            cg = cg_ref[...] / jnp.maximum(deng_ref[...], 1e-9)
            pre = jax.nn.relu(_dot(hg, wd3_ref[...]) +
                              _dot(ct, wd4_ref[0:d, :]) +
                              _dot(cg, wd4_ref[d:2 * d, :]) + bd2_ref[...])
            o_ref[...] = _dot(pre, ud_ref[...]) + bd3_ref[...]

    blk = lambda: pl.BlockSpec((b, d), lambda i: (i, 0))
    col = lambda: pl.BlockSpec((b, 1), lambda i: (i, 0))
    full = lambda r, c: pl.BlockSpec((r, c), lambda i: (0, 0))
    return pl.pallas_call(
        body,
        grid=(nb,),
        in_specs=[
            blk(), blk(),
            col(),
            pl.BlockSpec((1, 1, b), lambda i: (i, 0, 0)),
            col(), col(), full(1, M), full(1, M),
            full(M, d), full(M, 1), full(d, d), full(2 * d, d),
            full(1, d), full(d, 1), full(1, 1),
        ],
        out_specs=pl.BlockSpec((M, 1), lambda i: (0, 0)),
        out_shape=jax.ShapeDtypeStruct((M, 1), jnp.float32),
        scratch_shapes=[pltpu.VMEM((M, 1), jnp.float32),
                        pltpu.VMEM((M, 1), jnp.float32),
                        pltpu.VMEM((M, d), jnp.float32),
                        pltpu.VMEM((M, d), jnp.float32)],
    )(x_T, x_G, gid_col, gid_row, sc_t, sc_g, mx_t, mx_g,
      hgsum, cnt, wd3, wd4, bd2, ud, bd3)


def kernel(f, h, x_T, x_G, edge_index, graph_ids, wz, uz, bz, wr, ur, br,
           wg, ug, bg, wd1, wd2, bd1, a_t, a_g, wd3, wd4, bd2, ud, bd3):
    n, d = f.shape
    e = edge_index.shape[1]
    ns = 16
    k = 125
    nch = e // (ns * k)
    b = 2000
    src3 = edge_index[0].astype(jnp.int32).reshape(ns, nch // 16, 16, k)
    dst3 = edge_index[1].astype(jnp.int32).reshape(ns, nch // 16, 16, k)
    gid_col = graph_ids.astype(jnp.int32).reshape(n, 1)
    gid_row = graph_ids.astype(jnp.int32).reshape(n // b, 1, b)
    bd3_2d = bd3.reshape(1, 1)

    rh = _rh_kernel(f, h, wr, ur, br, n=n, d=d, b=b)
    s_arr, srh_arr = _sc_scatter(h, rh, src3, dst3, n=n, d=d, ns=ns,
                                 nch=nch, k=k)
    hgsum, cnt = _gates_kernel(f, s_arr, srh_arr, gid_row, wz, uz, bz,
                               wg, ug, bg, wd1, wd2, bd1, n=n, d=d, b=b)
    sc_t, sc_g, mx_t, mx_g = _scores_kernel(
        x_T, x_G, gid_col, gid_row, hgsum, cnt, a_t, a_g, n=n, d=d, b=b)
    return _pool_kernel(
        x_T, x_G, gid_col, gid_row, sc_t, sc_g, mx_t, mx_g,
        hgsum, cnt, wd3, wd4, bd2, ud, bd3_2d, n=n, d=d, b=b)


# speedup vs baseline: 11.8734x; 1.0366x over previous
import functools

import jax
import jax.numpy as jnp
from jax import lax
from jax.experimental import pallas as pl
from jax.experimental.pallas import tpu as pltpu
from jax.experimental.pallas import tpu_sc as plsc

M = 256
NEG = -3.0e38


def _sc_scatter(h, rh, src3, dst3, *, n, d, ns, nch, k):
    mesh = plsc.VectorSubcoreMesh(
        core_axis_name="c", subcore_axis_name="s", num_cores=2,
        num_subcores=ns)
    zc = 32
    np_ = ((n + ns * 128 - 1) // (ns * 128)) * ns * 128
    zr = np_ // ns
    nz = zr // zc
    sc_ = 16
    nsc = nch // sc_

    @functools.partial(
        pl.kernel,
        out_type=[jax.ShapeDtypeStruct((np_, d), jnp.float32),
                  jax.ShapeDtypeStruct((np_, d), jnp.float32)],
        mesh=mesh,
        scratch_types=[
            pltpu.VMEM((2, sc_, k), jnp.int32),
            pltpu.VMEM((2, sc_, k), jnp.int32),
            pltpu.VMEM((k, d), jnp.float32),
            pltpu.VMEM((k, d), jnp.float32),
            pltpu.VMEM((zc, d), jnp.float32),
            pltpu.VMEM_SHARED((np_, d), jnp.float32),
            pltpu.SemaphoreType.DMA,
            pltpu.SemaphoreType.DMA,
            pltpu.SemaphoreType.DMA,
        ],
    )
    def scatter_kernel(h_hbm, rh_hbm, src_hbm, dst_hbm, s_hbm, srh_hbm,
                       src_v, dst_v, rows0_v, rows1_v, zbuf_v, acc_sh,
                       sem0, sem1, isem):
        c = lax.axis_index("c")
        s = lax.axis_index("s")
        def zrow(t, carry):
            i = t // (d // 16)
            j = t % (d // 16)
            zbuf_v[i, pl.ds(j * 16, 16)] = jnp.zeros((16,), jnp.float32)
            return carry
        lax.fori_loop(0, zc * (d // 16), zrow, 0)
        def zacc(t, carry):
            pltpu.sync_copy(zbuf_v, acc_sh.at[pl.ds(s * zr + t * zc, zc)])
            return carry
        lax.fori_loop(0, nz, zacc, 0)
        plsc.subcore_barrier()

        bufs = (rows0_v, rows1_v)
        sems = (sem0, sem1)

        def edge_loop(vref):
            pltpu.sync_copy(src_hbm.at[s, 0], src_v.at[0])
            pltpu.sync_copy(dst_hbm.at[s, 0], dst_v.at[0])

            def super_step(t, carry):
                par = lax.rem(t, 2)

                @pl.when(t > 0)
                def _():
                    pltpu.make_async_copy(src_hbm.at[s, t], src_v.at[par],
                                          isem).wait()
                    pltpu.make_async_copy(dst_hbm.at[s, t], dst_v.at[par],
                                          isem).wait()

                @pl.when(t + 1 < nsc)
                def _():
                    pltpu.async_copy(src_hbm.at[s, t + 1], src_v.at[1 - par],
                                     isem)
                    pltpu.async_copy(dst_hbm.at[s, t + 1], dst_v.at[1 - par],
                                     isem)

                cur = pltpu.async_copy(vref.at[src_v.at[par, 0]], bufs[0],
                                       sems[0])
                for i in range(sc_):
                    nxt = None
                    if i + 1 < sc_:
                        j = (i + 1) % 2
                        nxt = pltpu.async_copy(
                            vref.at[src_v.at[par, i + 1]], bufs[j], sems[j])
                    cur.wait()
                    pltpu.sync_copy(bufs[i % 2], acc_sh.at[dst_v.at[par, i]],
                                    add=True)
                    cur = nxt
                return carry
            lax.fori_loop(0, nsc, super_step, 0)

        @pl.when(c == 0)
        def _():
            edge_loop(h_hbm)

        @pl.when(c == 1)
        def _():
            edge_loop(rh_hbm)

        plsc.subcore_barrier()

        @pl.when(c == 0)
        def _():
            pltpu.sync_copy(acc_sh.at[pl.ds(s * zr, zr)],
                            s_hbm.at[pl.ds(s * zr, zr)])

        @pl.when(c == 1)
        def _():
            pltpu.sync_copy(acc_sh.at[pl.ds(s * zr, zr)],
                            srh_hbm.at[pl.ds(s * zr, zr)])

    return scatter_kernel(h, rh, src3, dst3)


def _dot(a, b):
    return jax.lax.dot_general(a, b, (((1,), (0,)), ((), ())),
                               preferred_element_type=jnp.float32)


def _rh_kernel(f, h, wr, ur, br, *, n, d, b):
    def body(f_ref, h_ref, wr_ref, ur_ref, br_ref, o_ref):
        hb = h_ref[...]
        r = jax.nn.sigmoid(_dot(f_ref[...], wr_ref[...]) +
                           _dot(hb, ur_ref[...]) + br_ref[...])
        o_ref[...] = r * hb
    nb = n // b
    blk = lambda: pl.BlockSpec((b, d), lambda i: (i, 0))
    full = lambda r, c: pl.BlockSpec((r, c), lambda i: (0, 0))
    return pl.pallas_call(
        body,
        grid=(nb,),
        in_specs=[blk(), blk(), full(d, d), full(d, d), full(1, d)],
        out_specs=blk(),
        out_shape=jax.ShapeDtypeStruct((n, d), jnp.float32),
    )(f, h, wr, ur, br)


def _dot0(a, b):
    return jax.lax.dot_general(a, b, (((0,), (0,)), ((), ())),
                               preferred_element_type=jnp.float32)


def _graph_kernel(f, s_arr, srh_arr, x_T, x_G, gid_col, gid_row,
                  wz, uz, bz, wg, ug, bg, wd1, wd2, bd1, a_t, a_g,
                  wd3, wd4, bd2, ud, bd3, *, n, d, b):
    nb = n // b

    def body(f_ref, s_ref, srh_ref, xt_ref, xg_ref, gc_ref, gr_ref,
             wz_ref, uz_ref, bz_ref, wg_ref, ug_ref, bg_ref,
             wd1_ref, wd2_ref, bd1_ref, at_ref, ag_ref,
             wd3_ref, wd4_ref, bd2_ref, ud_ref, bd3_ref, o_ref,
             hgsum_ref, cnt_ref, qt_ref, qg_ref,
             mxt_ref, mxg_ref, dent_ref, deng_ref, ctT_ref, cgT_ref):
        ph = pl.program_id(0)
        i = pl.program_id(1)
        iota_col = jax.lax.broadcasted_iota(jnp.int32, (M, 1), 0)
        iota_row = jax.lax.broadcasted_iota(jnp.int32, (1, M), 1)
        oh = (gc_ref[...] == iota_row).astype(jnp.float32)
        ohT = (iota_col == gr_ref[0]).astype(jnp.float32)

        @pl.when(ph == 0)
        def _():
            fb = f_ref[...]
            sb = s_ref[...]
            z = jax.nn.sigmoid(_dot(fb, wz_ref[...]) + _dot(sb, uz_ref[...])
                               + bz_ref[...])
            ht = jnp.tanh(_dot(fb, wg_ref[...]) + _dot(srh_ref[...],
                                                       ug_ref[...])
                          + bg_ref[...])
            hn = (1.0 - z) * sb + z * ht
            hd = jax.nn.relu(_dot(fb, wd1_ref[...]) + _dot(hn, wd2_ref[...])
                             + bd1_ref[...])

            @pl.when(i == 0)
            def _():
                hgsum_ref[...] = jnp.zeros_like(hgsum_ref)
                cnt_ref[...] = jnp.zeros_like(cnt_ref)

            hgsum_ref[...] += _dot(ohT, hd)
            cnt_ref[...] += _dot(ohT, jnp.ones((b, 1), jnp.float32))

        @pl.when(ph == 1)
        def _():
            @pl.when(i == 0)
            def _():
                hg = hgsum_ref[...] / jnp.maximum(cnt_ref[...], 1.0)
                qt_ref[...] = _dot(hg, at_ref[...])
                qg_ref[...] = _dot(hg, ag_ref[...])
                mxt_ref[...] = jnp.full_like(mxt_ref, NEG)
                mxg_ref[...] = jnp.full_like(mxg_ref, NEG)
                dent_ref[...] = jnp.zeros_like(dent_ref)
                deng_ref[...] = jnp.zeros_like(deng_ref)
                ctT_ref[...] = jnp.zeros_like(ctT_ref)
                cgT_ref[...] = jnp.zeros_like(cgT_ref)

            def one(x_ref, q_ref, mx_ref, den_ref, cT_ref):
                xb = x_ref[...]
                sc = jnp.sum(xb * _dot(oh, q_ref[...]), axis=1,
                             keepdims=True)
                bm = jnp.max(jnp.where(oh > 0.5, sc, NEG), axis=0,
                             keepdims=True)
                m_new = jnp.maximum(mx_ref[...], bm)
                scale = jnp.exp(mx_ref[...] - m_new)
                mx_ref[...] = m_new
                mxg = jnp.sum(oh * m_new, axis=1, keepdims=True)
                e = jnp.exp(sc - mxg)
                eoh = oh * e
                den_ref[...] = (den_ref[...] * scale +
                                jnp.sum(eoh, axis=0, keepdims=True))
                cT_ref[...] = cT_ref[...] * scale + _dot0(xb, eoh)
            one(xt_ref, qt_ref, mxt_ref, dent_ref, ctT_ref)
            one(xg_ref, qg_ref, mxg_ref, deng_ref, cgT_ref)

            @pl.when(i == nb - 1)
            def _():
                hg = hgsum_ref[...] / jnp.maximum(cnt_ref[...], 1.0)
                ctn = ctT_ref[...] / jnp.maximum(dent_ref[...], 1e-9)
                cgn = cgT_ref[...] / jnp.maximum(deng_ref[...], 1e-9)
                pre = jax.nn.relu(_dot(hg, wd3_ref[...]) +
                                  _dot0(ctn, wd4_ref[0:d, :]) +
                                  _dot0(cgn, wd4_ref[d:2 * d, :]) +
                                  bd2_ref[...])
                o_ref[...] = _dot(pre, ud_ref[...]) + bd3_ref[...]

    p0 = lambda ph, i: (jnp.where(ph == 0, i, 0), 0)
    p1 = lambda ph, i: (jnp.where(ph == 0, 0, i), 0)
    full = lambda r, c: pl.BlockSpec((r, c), lambda ph, i: (0, 0))
    return pl.pallas_call(
        body,
        grid=(2, nb),
        in_specs=[
            pl.BlockSpec((b, d), p0),
            pl.BlockSpec((b, d), p0),
            pl.BlockSpec((b, d), p0),
            pl.BlockSpec((b, d), p1),
            pl.BlockSpec((b, d), p1),
            pl.BlockSpec((b, 1), lambda ph, i: (i, 0)),
            pl.BlockSpec((1, 1, b), lambda ph, i: (i, 0, 0)),
            full(d, d), full(d, d), full(1, d),
            full(d, d), full(d, d), full(1, d),
            full(d, d), full(d, d), full(1, d),
            full(d, d), full(d, d),
            full(d, d), full(2 * d, d), full(1, d), full(d, 1), full(1, 1),
        ],
        out_specs=pl.BlockSpec((M, 1), lambda ph, i: (0, 0)),
        out_shape=jax.ShapeDtypeStruct((M, 1), jnp.float32),
        scratch_shapes=[
            pltpu.VMEM((M, d), jnp.float32),
            pltpu.VMEM((M, 1), jnp.float32),
            pltpu.VMEM((M, d), jnp.float32),
            pltpu.VMEM((M, d), jnp.float32),
            pltpu.VMEM((1, M), jnp.float32),
            pltpu.VMEM((1, M), jnp.float32),
            pltpu.VMEM((1, M), jnp.float32),
            pltpu.VMEM((1, M), jnp.float32),
            pltpu.VMEM((d, M), jnp.float32),
            pltpu.VMEM((d, M), jnp.float32),
        ],
    )(f, s_arr, srh_arr, x_T, x_G, gid_col, gid_row,
      wz, uz, bz, wg, ug, bg, wd1, wd2, bd1, a_t, a_g,
      wd3, wd4, bd2, ud, bd3)


def kernel(f, h, x_T, x_G, edge_index, graph_ids, wz, uz, bz, wr, ur, br,
           wg, ug, bg, wd1, wd2, bd1, a_t, a_g, wd3, wd4, bd2, ud, bd3):
    n, d = f.shape
    e = edge_index.shape[1]
    ns = 16
    k = 125
    nch = e // (ns * k)
    b = 2000
    src3 = edge_index[0].astype(jnp.int32).reshape(ns, nch // 16, 16, k)
    dst3 = edge_index[1].astype(jnp.int32).reshape(ns, nch // 16, 16, k)
    gid_col = graph_ids.astype(jnp.int32).reshape(n, 1)
    gid_row = graph_ids.astype(jnp.int32).reshape(n // b, 1, b)
    bd3_2d = bd3.reshape(1, 1)

    rh = _rh_kernel(f, h, wr, ur, br, n=n, d=d, b=b)
    s_arr, srh_arr = _sc_scatter(h, rh, src3, dst3, n=n, d=d, ns=ns,
                                 nch=nch, k=k)
    return _graph_kernel(f, s_arr, srh_arr, x_T, x_G, gid_col, gid_row,
                         wz, uz, bz, wg, ug, bg, wd1, wd2, bd1, a_t, a_g,
                         wd3, wd4, bd2, ud, bd3_2d, n=n, d=d, b=b)
